# baseline (speedup 1.0000x reference)
import functools

import jax
import jax.numpy as jnp
from jax import lax
from jax.experimental import pallas as pl
from jax.experimental.pallas import tpu as pltpu
from jax.experimental.pallas import tpu_sc as plsc

N = 10000
D = 128
E = 320000

NC = 2
NS = 16
K = 64
NB = 320
EPT = NB * K
E_PAD = NS * EPT
ACC_ROWS = 10112
ZK = 64
ZCHUNKS = ACC_ROWS // ZK
CH = 32
NCH = NB // CH
OFULL = N // ZK
OTAIL = N - OFULL * ZK

_TC_BLK = 1000


def _prep_body(x_ref, tab_ref):
    g = jnp.maximum(x_ref[...], 0.0) + 1e-07
    eg = jnp.exp(g)
    tab_ref[0] = eg
    tab_ref[1] = eg * g


def _prep(x):
    return pl.pallas_call(
        _prep_body,
        grid=(N // _TC_BLK,),
        in_specs=[pl.BlockSpec((_TC_BLK, D), lambda i: (i, 0))],
        out_specs=pl.BlockSpec((2, _TC_BLK, D), lambda i: (0, i, 0)),
        out_shape=jax.ShapeDtypeStruct((2, N, D), jnp.float32),
    )(x)


def _sc_edge_body(tab_hbm, srcs_hbm, dsts_hbm, zeros_hbm, out_hbm,
                  acc, idxs_v, idxd_v, rows2_v, gsem, ssem, isem, osem):
    c = lax.axis_index("c")
    s = lax.axis_index("s")

    pltpu.async_copy(srcs_hbm.at[c, s, pl.ds(0, CH)], idxs_v.at[0], isem)
    pltpu.async_copy(dsts_hbm.at[s, pl.ds(0, CH)], idxd_v.at[0], isem)

    pltpu.sync_copy(zeros_hbm, rows2_v.at[0])

    def zbody(k, carry):
        ch = s + k * NS

        @pl.when(ch < ZCHUNKS)
        def _():
            pltpu.sync_copy(rows2_v.at[0], acc.at[pl.ds(ch * ZK, ZK)])

        return carry

    lax.fori_loop(0, (ZCHUNKS + NS - 1) // NS, zbody, 0)
    pltpu.make_async_copy(srcs_hbm.at[c, s, pl.ds(0, CH)],
                          idxs_v.at[0], isem).wait()
    pltpu.make_async_copy(dsts_hbm.at[s, pl.ds(0, CH)],
                          idxd_v.at[0], isem).wait()
    plsc.subcore_barrier()

    for gp in range(3):
        pltpu.async_copy(tab_hbm.at[idxs_v.at[0, gp]], rows2_v.at[gp], gsem)

    def body(g, carry):
        par = lax.rem(g, 4)
        sl = lax.rem(lax.div(g, CH), 2)
        jj = lax.rem(g, CH)
        g3 = g + 3
        sl3 = lax.rem(lax.div(g3, CH), 2)
        jj3 = lax.rem(g3, CH)

        @pl.when(g > 0)
        def _():
            gm = g - 1
            pltpu.make_async_copy(
                rows2_v.at[lax.rem(gm, 4)],
                acc.at[idxd_v.at[lax.rem(lax.div(gm, CH), 2),
                                 lax.rem(gm, CH)]],
                ssem).wait()

        @pl.when(jnp.logical_and(jj == 0, g + CH < NB))
        def _():
            nxt = (lax.div(g, CH) + 1) * CH
            pltpu.async_copy(srcs_hbm.at[c, s, pl.ds(nxt, CH)],
                             idxs_v.at[1 - sl], isem)
            pltpu.async_copy(dsts_hbm.at[s, pl.ds(nxt, CH)],
                             idxd_v.at[1 - sl], isem)

        @pl.when(jnp.logical_and(jj == CH - 3, g3 < NB))
        def _():
            nxt = (lax.div(g, CH) + 1) * CH
            pltpu.make_async_copy(srcs_hbm.at[c, s, pl.ds(nxt, CH)],
                                  idxs_v.at[1 - sl], isem).wait()
            pltpu.make_async_copy(dsts_hbm.at[s, pl.ds(nxt, CH)],
                                  idxd_v.at[1 - sl], isem).wait()

        @pl.when(g3 < NB)
        def _():
            pltpu.async_copy(tab_hbm.at[idxs_v.at[sl3, jj3]],
                             rows2_v.at[lax.rem(g3, 4)], gsem)

        pltpu.make_async_copy(tab_hbm.at[idxs_v.at[sl, jj]],
                              rows2_v.at[par], gsem).wait()
        pltpu.async_copy(rows2_v.at[par], acc.at[idxd_v.at[sl, jj]], ssem,
                         add=True)
        return carry

    lax.fori_loop(0, NB, body, 0)
    pltpu.make_async_copy(
        rows2_v.at[lax.rem(NB - 1, 4)],
        acc.at[idxd_v.at[lax.rem(lax.div(NB - 1, CH), 2), CH - 1]],
        ssem).wait()
    plsc.subcore_barrier()

    def obody(k, carry):
        ch = s + k * NS

        @pl.when(ch < OFULL)
        def _():
            @pl.when(k > 0)
            def _():
                pltpu.make_async_copy(
                    rows2_v.at[0],
                    out_hbm.at[c, pl.ds((s + (k - 1) * NS) * ZK, ZK)],
                    osem).wait()

            pltpu.sync_copy(acc.at[pl.ds(ch * ZK, ZK)], rows2_v.at[0])
            pltpu.async_copy(rows2_v.at[0], out_hbm.at[c, pl.ds(ch * ZK, ZK)],
                             osem)

        return carry

    lax.fori_loop(0, (OFULL + NS - 1) // NS, obody, 0)
    pltpu.make_async_copy(rows2_v.at[0], out_hbm.at[c, pl.ds(s * ZK, ZK)],
                          osem).wait()

    @pl.when(s == 0)
    def _():
        pltpu.sync_copy(acc.at[pl.ds(OFULL * ZK, OTAIL)],
                        rows2_v.at[0, pl.ds(0, OTAIL)])
        pltpu.sync_copy(rows2_v.at[0, pl.ds(0, OTAIL)],
                        out_hbm.at[c, pl.ds(OFULL * ZK, OTAIL)])


_sc_edge = functools.partial(
    pl.kernel,
    out_type=jax.ShapeDtypeStruct((2, N, D), jnp.float32),
    mesh=plsc.VectorSubcoreMesh(core_axis_name="c", subcore_axis_name="s"),
    scratch_types=[
        pltpu.VMEM_SHARED((ACC_ROWS, D), jnp.float32),
        pltpu.VMEM((2, CH, K), jnp.int32),
        pltpu.VMEM((2, CH, K), jnp.int32),
        pltpu.VMEM((4, K, D), jnp.float32),
        pltpu.SemaphoreType.DMA,
        pltpu.SemaphoreType.DMA,
        pltpu.SemaphoreType.DMA,
        pltpu.SemaphoreType.DMA,
    ],
)(_sc_edge_body)


def _final_body(acc_ref, x_ref, wt_ref, b_ref, out_ref):
    m = acc_ref[1] / (acc_ref[0] + 1e-16)
    feats = x_ref[...] + m
    out_ref[...] = (
        jnp.dot(feats, wt_ref[...], preferred_element_type=jnp.float32)
        + b_ref[...]
    )


def _final(acc, x, wt, b2):
    return pl.pallas_call(
        _final_body,
        grid=(N // _TC_BLK,),
        in_specs=[
            pl.BlockSpec((2, _TC_BLK, D), lambda i: (0, i, 0)),
            pl.BlockSpec((_TC_BLK, D), lambda i: (i, 0)),
            pl.BlockSpec((D, D), lambda i: (0, 0)),
            pl.BlockSpec((1, D), lambda i: (0, 0)),
        ],
        out_specs=pl.BlockSpec((_TC_BLK, D), lambda i: (i, 0)),
        out_shape=jax.ShapeDtypeStruct((N, D), jnp.float32),
    )(acc, x, wt, b2)


def kernel(x, edge_index, W, b):
    src = edge_index[0]
    dst = edge_index[1]
    pad = E_PAD - E
    src_p = jnp.concatenate([src, jnp.zeros((pad,), jnp.int32)])
    dst_p = jnp.concatenate([dst, jnp.full((pad,), N, jnp.int32)])
    src3 = src_p.reshape(NS, NB, K)
    srcs = jnp.stack([src3, src3 + N])
    dsts = dst_p.reshape(NS, NB, K)
    zeros = jnp.zeros((ZK, D), jnp.float32)

    tab = _prep(x).reshape(2 * N, D)
    acc = _sc_edge(tab, srcs, dsts, zeros)
    return _final(acc, x, W.T, b.reshape(1, D))


# speedup vs baseline: 2.6651x; 2.6651x over previous
import functools

import jax
import jax.numpy as jnp
from jax import lax
from jax.experimental import pallas as pl
from jax.experimental.pallas import tpu as pltpu
from jax.experimental.pallas import tpu_sc as plsc

N = 10000
D = 128
E = 320000

NC = 2
NS = 16
K = 125
NB = 160
ACC_ROWS = 10112
ZK = 64
ZCHUNKS = ACC_ROWS // ZK
CH = 16
OFULL = N // ZK
OTAIL = N - OFULL * ZK

_TC_BLK = 1000


def _prep_body(x_ref, tab_ref):
    g = jnp.maximum(x_ref[...], 0.0) + 1e-07
    eg = jnp.exp(g)
    tab_ref[0] = eg
    tab_ref[1] = eg * g


def _prep(x):
    return pl.pallas_call(
        _prep_body,
        grid=(N // _TC_BLK,),
        in_specs=[pl.BlockSpec((_TC_BLK, D), lambda i: (i, 0))],
        out_specs=pl.BlockSpec((2, _TC_BLK, D), lambda i: (0, i, 0)),
        out_shape=jax.ShapeDtypeStruct((2, N, D), jnp.float32),
    )(x)


def _sc_edge_body(tab_hbm, srcs_hbm, dsts_hbm, zeros_hbm, out_hbm,
                  acc, idxs_v, idxd_v, rows2_v, gsem, ssem, isem, osem):
    c = lax.axis_index("c")
    s = lax.axis_index("s")

    pltpu.async_copy(srcs_hbm.at[c, s, pl.ds(0, CH)], idxs_v.at[0], isem)
    pltpu.async_copy(dsts_hbm.at[s, pl.ds(0, CH)], idxd_v.at[0], isem)

    pltpu.sync_copy(zeros_hbm, rows2_v.at[0, pl.ds(0, ZK)])

    def zbody(k, carry):
        ch = s + k * NS

        @pl.when(ch < ZCHUNKS)
        def _():
            pltpu.sync_copy(rows2_v.at[0, pl.ds(0, ZK)],
                            acc.at[pl.ds(ch * ZK, ZK)])

        return carry

    lax.fori_loop(0, (ZCHUNKS + NS - 1) // NS, zbody, 0)
    pltpu.make_async_copy(srcs_hbm.at[c, s, pl.ds(0, CH)],
                          idxs_v.at[0], isem).wait()
    pltpu.make_async_copy(dsts_hbm.at[s, pl.ds(0, CH)],
                          idxd_v.at[0], isem).wait()
    plsc.subcore_barrier()

    pltpu.async_copy(tab_hbm.at[idxs_v.at[0, 0]], rows2_v.at[0], gsem)

    def body(g, carry):
        par = lax.rem(g, 2)
        sl = lax.rem(lax.div(g, CH), 2)
        jj = lax.rem(g, CH)
        g1 = g + 1
        sl1 = lax.rem(lax.div(g1, CH), 2)
        jj1 = lax.rem(g1, CH)

        @pl.when(g > 0)
        def _():
            gm = g - 1
            pltpu.make_async_copy(
                rows2_v.at[1 - par],
                acc.at[idxd_v.at[lax.rem(lax.div(gm, CH), 2),
                                 lax.rem(gm, CH)]],
                ssem).wait()

        @pl.when(jnp.logical_and(jj == 0, g + CH < NB))
        def _():
            nxt = (lax.div(g, CH) + 1) * CH
            pltpu.async_copy(srcs_hbm.at[c, s, pl.ds(nxt, CH)],
                             idxs_v.at[1 - sl], isem)
            pltpu.async_copy(dsts_hbm.at[s, pl.ds(nxt, CH)],
                             idxd_v.at[1 - sl], isem)

        @pl.when(jnp.logical_and(jj == CH - 1, g1 < NB))
        def _():
            nxt = (lax.div(g, CH) + 1) * CH
            pltpu.make_async_copy(srcs_hbm.at[c, s, pl.ds(nxt, CH)],
                                  idxs_v.at[1 - sl], isem).wait()
            pltpu.make_async_copy(dsts_hbm.at[s, pl.ds(nxt, CH)],
                                  idxd_v.at[1 - sl], isem).wait()

        @pl.when(g1 < NB)
        def _():
            pltpu.async_copy(tab_hbm.at[idxs_v.at[sl1, jj1]],
                             rows2_v.at[1 - par], gsem)

        pltpu.make_async_copy(tab_hbm.at[idxs_v.at[sl, jj]],
                              rows2_v.at[par], gsem).wait()
        pltpu.async_copy(rows2_v.at[par], acc.at[idxd_v.at[sl, jj]], ssem,
                         add=True)
        return carry

    lax.fori_loop(0, NB, body, 0)
    pltpu.make_async_copy(
        rows2_v.at[lax.rem(NB - 1, 2)],
        acc.at[idxd_v.at[lax.rem(lax.div(NB - 1, CH), 2), CH - 1]],
        ssem).wait()
    plsc.subcore_barrier()

    def obody(k, carry):
        ch = s + k * NS

        @pl.when(ch < OFULL)
        def _():
            @pl.when(k > 0)
            def _():
                pltpu.make_async_copy(
                    rows2_v.at[0, pl.ds(0, ZK)],
                    out_hbm.at[c, pl.ds((s + (k - 1) * NS) * ZK, ZK)],
                    osem).wait()

            pltpu.sync_copy(acc.at[pl.ds(ch * ZK, ZK)],
                            rows2_v.at[0, pl.ds(0, ZK)])
            pltpu.async_copy(rows2_v.at[0, pl.ds(0, ZK)],
                             out_hbm.at[c, pl.ds(ch * ZK, ZK)], osem)

        return carry

    lax.fori_loop(0, (OFULL + NS - 1) // NS, obody, 0)
    pltpu.make_async_copy(rows2_v.at[0, pl.ds(0, ZK)],
                          out_hbm.at[c, pl.ds(s * ZK, ZK)], osem).wait()

    @pl.when(s == 0)
    def _():
        pltpu.sync_copy(acc.at[pl.ds(OFULL * ZK, OTAIL)],
                        rows2_v.at[0, pl.ds(0, OTAIL)])
        pltpu.sync_copy(rows2_v.at[0, pl.ds(0, OTAIL)],
                        out_hbm.at[c, pl.ds(OFULL * ZK, OTAIL)])


_sc_edge = functools.partial(
    pl.kernel,
    out_type=jax.ShapeDtypeStruct((2, N, D), jnp.float32),
    mesh=plsc.VectorSubcoreMesh(core_axis_name="c", subcore_axis_name="s"),
    scratch_types=[
        pltpu.VMEM_SHARED((ACC_ROWS, D), jnp.float32),
        pltpu.VMEM((2, CH, K), jnp.int32),
        pltpu.VMEM((2, CH, K), jnp.int32),
        pltpu.VMEM((2, K, D), jnp.float32),
        pltpu.SemaphoreType.DMA,
        pltpu.SemaphoreType.DMA,
        pltpu.SemaphoreType.DMA,
        pltpu.SemaphoreType.DMA,
    ],
)(_sc_edge_body)


def _final_body(acc_ref, x_ref, wt_ref, b_ref, out_ref):
    m = acc_ref[1] / (acc_ref[0] + 1e-16)
    feats = x_ref[...] + m
    out_ref[...] = (
        jnp.dot(feats, wt_ref[...], preferred_element_type=jnp.float32)
        + b_ref[...]
    )


def _final(acc, x, wt, b2):
    return pl.pallas_call(
        _final_body,
        grid=(N // _TC_BLK,),
        in_specs=[
            pl.BlockSpec((2, _TC_BLK, D), lambda i: (0, i, 0)),
            pl.BlockSpec((_TC_BLK, D), lambda i: (i, 0)),
            pl.BlockSpec((D, D), lambda i: (0, 0)),
            pl.BlockSpec((1, D), lambda i: (0, 0)),
        ],
        out_specs=pl.BlockSpec((_TC_BLK, D), lambda i: (i, 0)),
        out_shape=jax.ShapeDtypeStruct((N, D), jnp.float32),
    )(acc, x, wt, b2)


def kernel(x, edge_index, W, b):
    src3 = edge_index[0].reshape(NS, NB, K)
    srcs = jnp.stack([src3, src3 + N])
    dsts = edge_index[1].reshape(NS, NB, K)
    zeros = jnp.zeros((ZK, D), jnp.float32)

    tab = _prep(x).reshape(2 * N, D)
    acc = _sc_edge(tab, srcs, dsts, zeros)
    return _final(acc, x, W.T, b.reshape(1, D))
